# baseline (speedup 1.0000x reference)
import functools

import jax
import jax.numpy as jnp
from jax import lax
from jax.experimental import pallas as pl
from jax.experimental.pallas import tpu as pltpu
from jax.experimental.pallas import tpu_sc as plsc

N = 10000
E = 160000
D_IN = 128
H = 64

NC = 2
NS = 16
CH = 64
E_PAD = 163840
NCH = E_PAD // (NS * CH)
HCH = NCH // 2
RING = 8
NPAD = 10240
ZCH = NPAD // NS // CH
TST = N // NS
TROWS = 2 * N

BLK = 10000


def _prep_edges(edge_index_1, edge_index_2):
    pad = E_PAD - E
    src_idx = jnp.pad(
        jnp.stack([edge_index_1[0], edge_index_2[0]]),
        ((0, 0), (0, pad))).reshape(NC * NS, NCH, CH)
    dst_idx = jnp.pad(
        jnp.stack([edge_index_1[1], edge_index_2[1]]),
        ((0, 0), (0, pad)), constant_values=N).reshape(NC * NS, NCH, CH)
    return src_idx, dst_idx


def _sc_propagate(table, src_idx, dst_idx, zstripe):
    mesh = plsc.VectorSubcoreMesh(core_axis_name="c", subcore_axis_name="s")

    @functools.partial(
        pl.kernel,
        out_type=jax.ShapeDtypeStruct((NC * NPAD, H), jnp.float32),
        mesh=mesh,
        scratch_types=[
            pltpu.VMEM((HCH, CH), jnp.int32),
            pltpu.VMEM((HCH, CH), jnp.int32),
            [pltpu.VMEM((CH, H), jnp.float32)] * RING,
            pltpu.VMEM_SHARED((N, H), jnp.float32),
            pltpu.VMEM_SHARED((NPAD, H), jnp.float32),
            [pltpu.SemaphoreType.DMA] * RING,
            [pltpu.SemaphoreType.DMA] * RING,
        ],
        compiler_params=pltpu.CompilerParams(use_tc_tiling_on_sc=False),
    )
    def prop(table_hbm, src_hbm, dst_hbm, z_hbm, out_hbm, src_v, dst_v,
             rows, tab_sh, acc_sh, gsem, ssem):
        c = lax.axis_index("c")
        s = lax.axis_index("s")
        w = c * NS + s
        zbase = s * (NPAD // NS)

        cz = pltpu.async_copy(z_hbm, acc_sh.at[pl.ds(zbase, NPAD // NS)],
                              ssem[0])
        ct = pltpu.async_copy(table_hbm.at[pl.ds(c * N + s * TST, TST)],
                              tab_sh.at[pl.ds(s * TST, TST)], ssem[1])

        def gissue(j, k):
            pltpu.async_copy(tab_sh.at[src_v.at[j]], rows[k], gsem[k])

        def gwait(j, k):
            pltpu.make_async_copy(
                tab_sh.at[src_v.at[j]], rows[k], gsem[k]).wait()

        def sissue(j, k):
            pltpu.async_copy(rows[k], acc_sh.at[dst_v.at[j]], ssem[k],
                             add=True)

        def swait(j, k):
            pltpu.make_async_copy(
                rows[k], acc_sh.at[dst_v.at[j]], ssem[k]).wait()

        def run_half(h):
            if h == 0:
                ci1 = pltpu.async_copy(
                    src_hbm.at[w, pl.ds(0, HCH)], src_v, ssem[2])
                ci2 = pltpu.async_copy(
                    dst_hbm.at[w, pl.ds(0, HCH)], dst_v, ssem[3])
                cz.wait()
                ct.wait()
                ci1.wait()
                ci2.wait()
                plsc.subcore_barrier()
            else:
                pltpu.sync_copy(src_hbm.at[w, pl.ds(h * HCH, HCH)], src_v)
                pltpu.sync_copy(dst_hbm.at[w, pl.ds(h * HCH, HCH)], dst_v)
            lead = RING // 2
            for t in range(lead):
                gissue(t, t)

            def stepn(i, carry):
                for kk in range(RING):
                    j = RING * i + kk
                    m = (kk + lead) % RING
                    gwait(j, kk)
                    sissue(j, kk)

                    @pl.when(j + lead < HCH)
                    def _():
                        @pl.when(j >= lead)
                        def _():
                            swait(j - lead, m)
                        gissue(j + lead, m)
                return carry
            lax.fori_loop(0, HCH // RING, stepn, 0)
            for t in range(RING):
                swait(HCH - RING + t, t)

        run_half(0)
        run_half(1)
        plsc.subcore_barrier()

        pltpu.sync_copy(
            acc_sh.at[pl.ds(zbase, NPAD // NS)],
            out_hbm.at[pl.ds(c * NPAD + zbase, NPAD // NS)])

    return prop(table, src_idx, dst_idx, zstripe)


def _tc_project_l1(x, w_stack):
    def body(x_ref, w_ref, o_ref):
        xv = x_ref[...]
        o_ref[0] = jnp.dot(xv, w_ref[0], preferred_element_type=jnp.float32)
        o_ref[1] = jnp.dot(xv, w_ref[1], preferred_element_type=jnp.float32)

    out = pl.pallas_call(
        body,
        out_shape=jax.ShapeDtypeStruct((2, N, H), jnp.float32),
    )(x, w_stack)
    return out.reshape(TROWS, H)


def _tc_combine_project(parts, w_stack):
    def body(p_ref, w_ref, o_ref):
        h = jnp.maximum(p_ref[0, :N] + p_ref[1, :N], 0.0)
        o_ref[0] = jnp.dot(h, w_ref[0], preferred_element_type=jnp.float32)
        o_ref[1] = jnp.dot(h, w_ref[1], preferred_element_type=jnp.float32)

    out = pl.pallas_call(
        body,
        out_shape=jax.ShapeDtypeStruct((2, N, H), jnp.float32),
    )(parts, w_stack)
    return out.reshape(TROWS, H)


def _tc_combine(parts):
    def body(p_ref, o_ref):
        o_ref[...] = jnp.maximum(p_ref[0, :N] + p_ref[1, :N], 0.0)

    return pl.pallas_call(
        body,
        out_shape=jax.ShapeDtypeStruct((N, H), jnp.float32),
    )(parts)


def kernel(x, edge_index_1, edge_index_2, W1_1, W1_2, W2_1, W2_2):
    src_idx, dst_idx = _prep_edges(edge_index_1, edge_index_2)
    zstripe = jnp.zeros((NPAD // NS, H), jnp.float32)

    table1 = _tc_project_l1(x, jnp.stack([W1_1, W1_2]))
    parts1 = _sc_propagate(
        table1, src_idx, dst_idx, zstripe).reshape(NC, NPAD, H)
    table2 = _tc_combine_project(parts1, jnp.stack([W2_1, W2_2]))
    parts2 = _sc_propagate(
        table2, src_idx, dst_idx, zstripe).reshape(NC, NPAD, H)
    return _tc_combine(parts2)


# speedup vs baseline: 1.0044x; 1.0044x over previous
import functools

import jax
import jax.numpy as jnp
from jax import lax
from jax.experimental import pallas as pl
from jax.experimental.pallas import tpu as pltpu
from jax.experimental.pallas import tpu_sc as plsc

N = 10000
E = 160000
D_IN = 128
H = 64

NC = 2
NS = 16
CH = 64
E_PAD = 163840
NCH = E_PAD // (NS * CH)
HCH = NCH // 2
RING = 8
NPAD = 10240
TST = N // NS
TROWS = 2 * N

BLK = 10000


def _prep_edges(edge_index_1, edge_index_2):
    pad = E_PAD - E
    src_idx = jnp.pad(
        jnp.stack([edge_index_1[0], edge_index_2[0]]),
        ((0, 0), (0, pad))).reshape(NC * NS, NCH, CH)
    dst_idx = jnp.pad(
        jnp.stack([edge_index_1[1], edge_index_2[1]]),
        ((0, 0), (0, pad)), constant_values=N).reshape(NC * NS, NCH, CH)
    return src_idx, dst_idx


def _sc_propagate(table, src_idx, dst_idx, zstripe):
    mesh = plsc.VectorSubcoreMesh(core_axis_name="c", subcore_axis_name="s")

    @functools.partial(
        pl.kernel,
        out_type=jax.ShapeDtypeStruct((NC * NPAD, H), jnp.float32),
        mesh=mesh,
        scratch_types=[
            pltpu.VMEM((HCH, CH), jnp.int32),
            pltpu.VMEM((HCH, CH), jnp.int32),
            [pltpu.VMEM((CH, H), jnp.float32)] * RING,
            pltpu.VMEM_SHARED((N, H), jnp.float32),
            pltpu.VMEM_SHARED((NPAD, H), jnp.float32),
            [pltpu.SemaphoreType.DMA] * RING,
            [pltpu.SemaphoreType.DMA] * RING,
        ],
        compiler_params=pltpu.CompilerParams(use_tc_tiling_on_sc=False),
    )
    def prop(table_hbm, src_hbm, dst_hbm, z_hbm, out_hbm, src_v, dst_v,
             rows, tab_sh, acc_sh, gsem, ssem):
        c = lax.axis_index("c")
        s = lax.axis_index("s")
        w = c * NS + s
        zbase = s * (NPAD // NS)

        cz = pltpu.async_copy(z_hbm, acc_sh.at[pl.ds(zbase, NPAD // NS)],
                              ssem[0])
        ct = pltpu.async_copy(table_hbm.at[pl.ds(c * N + s * TST, TST)],
                              tab_sh.at[pl.ds(s * TST, TST)], ssem[1])

        def gissue(j, k):
            pltpu.async_copy(tab_sh.at[src_v.at[j]], rows[k], gsem[k])

        def gwait(j, k):
            pltpu.make_async_copy(
                tab_sh.at[src_v.at[j]], rows[k], gsem[k]).wait()

        def sissue(j, k):
            pltpu.async_copy(rows[k], acc_sh.at[dst_v.at[j]], ssem[k],
                             add=True)

        def swait(j, k):
            pltpu.make_async_copy(
                rows[k], acc_sh.at[dst_v.at[j]], ssem[k]).wait()

        def run_half(h):
            if h == 0:
                ci1 = pltpu.async_copy(
                    src_hbm.at[w, pl.ds(0, HCH)], src_v, ssem[2])
                ci2 = pltpu.async_copy(
                    dst_hbm.at[w, pl.ds(0, HCH)], dst_v, ssem[3])
                cz.wait()
                ct.wait()
                ci1.wait()
                ci2.wait()
                plsc.subcore_barrier()
            else:
                pltpu.sync_copy(src_hbm.at[w, pl.ds(h * HCH, HCH)], src_v)
                pltpu.sync_copy(dst_hbm.at[w, pl.ds(h * HCH, HCH)], dst_v)
            lead = RING // 2
            for t in range(lead):
                gissue(t, t)

            def stepn(i, carry):
                for kk in range(RING):
                    j = RING * i + kk
                    m = (kk + lead) % RING
                    gwait(j, kk)
                    sissue(j, kk)

                    @pl.when(j + lead < HCH)
                    def _():
                        @pl.when(j >= lead)
                        def _():
                            swait(j - lead, m)
                        gissue(j + lead, m)
                return carry
            lax.fori_loop(0, HCH // RING, stepn, 0)
            for t in range(RING):
                swait(HCH - RING + t, t)

        run_half(0)
        run_half(1)
        plsc.subcore_barrier()

        pltpu.sync_copy(
            acc_sh.at[pl.ds(zbase, NPAD // NS)],
            out_hbm.at[pl.ds(c * NPAD + zbase, NPAD // NS)])

    return prop(table, src_idx, dst_idx, zstripe)


def _tc_project_l1(x, w_stack):
    def body(x_ref, w_ref, o_ref):
        o_ref[0] = jnp.dot(x_ref[...], w_ref[0],
                           preferred_element_type=jnp.float32)

    out = pl.pallas_call(
        body,
        grid=(N // BLK, 2),
        in_specs=[
            pl.BlockSpec((BLK, D_IN), lambda i, j: (i, 0)),
            pl.BlockSpec((1, D_IN, H), lambda i, j: (j, 0, 0)),
        ],
        out_specs=pl.BlockSpec((1, BLK, H), lambda i, j: (j, i, 0)),
        out_shape=jax.ShapeDtypeStruct((2, N, H), jnp.float32),
    )(x, w_stack)
    return out.reshape(TROWS, H)


def _tc_combine_project(parts, w_stack):
    def body(a_ref, b_ref, w_ref, o_ref):
        h = jnp.maximum(a_ref[0] + b_ref[0], 0.0)
        o_ref[0] = jnp.dot(h, w_ref[0], preferred_element_type=jnp.float32)

    p0_spec = pl.BlockSpec((1, BLK, H), lambda i, j: (0, i, 0))
    p1_spec = pl.BlockSpec((1, BLK, H), lambda i, j: (1, i, 0))
    out = pl.pallas_call(
        body,
        grid=(N // BLK, 2),
        in_specs=[p0_spec, p1_spec,
                  pl.BlockSpec((1, H, H), lambda i, j: (j, 0, 0))],
        out_specs=pl.BlockSpec((1, BLK, H), lambda i, j: (j, i, 0)),
        out_shape=jax.ShapeDtypeStruct((2, N, H), jnp.float32),
    )(parts, parts, w_stack)
    return out.reshape(TROWS, H)


def _tc_combine(parts):
    def body(a_ref, b_ref, o_ref):
        o_ref[...] = jnp.maximum(a_ref[0] + b_ref[0], 0.0)

    p0_spec = pl.BlockSpec((1, BLK, H), lambda i: (0, i, 0))
    p1_spec = pl.BlockSpec((1, BLK, H), lambda i: (1, i, 0))
    return pl.pallas_call(
        body,
        grid=(N // BLK,),
        in_specs=[p0_spec, p1_spec],
        out_specs=pl.BlockSpec((BLK, H), lambda i: (i, 0)),
        out_shape=jax.ShapeDtypeStruct((N, H), jnp.float32),
    )(parts, parts)


def kernel(x, edge_index_1, edge_index_2, W1_1, W1_2, W2_1, W2_2):
    src_idx, dst_idx = _prep_edges(edge_index_1, edge_index_2)
    zstripe = jnp.zeros((NPAD // NS, H), jnp.float32)

    table1 = _tc_project_l1(x, jnp.stack([W1_1, W1_2]))
    parts1 = _sc_propagate(
        table1, src_idx, dst_idx, zstripe).reshape(NC, NPAD, H)
    table2 = _tc_combine_project(parts1, jnp.stack([W2_1, W2_2]))
    parts2 = _sc_propagate(
        table2, src_idx, dst_idx, zstripe).reshape(NC, NPAD, H)
    return _tc_combine(parts2)
